# Initial kernel scaffold; baseline (speedup 1.0000x reference)
import jax
import jax.numpy as jnp
from jax.experimental import pallas as pl


def kernel(x, edge_index, W1, b1, W2, b2, Wl, bl):
    raise NotImplementedError("write your pallas kernel here")


# speedup vs baseline: 21.5258x; 21.5258x over previous
import functools

import jax
import jax.numpy as jnp
from jax import lax
from jax.experimental import pallas as pl
from jax.experimental.pallas import tpu as pltpu
from jax.experimental.pallas import tpu_sc as plsc

NC = 2
NS = 16
K = 128
G = 8


def _sc_degree(dst_p, ndeg):
    C = dst_p.shape[2]
    per = ndeg // NS
    mesh = plsc.VectorSubcoreMesh(core_axis_name="c", subcore_axis_name="s")

    @functools.partial(
        pl.kernel,
        out_type=jax.ShapeDtypeStruct((NC, 1, ndeg), jnp.float32),
        mesh=mesh,
        scratch_types=[
            pltpu.VMEM((G, K), jnp.int32),
            pltpu.VMEM((K,), jnp.float32),
            pltpu.VMEM((per,), jnp.float32),
            pltpu.VMEM_SHARED((ndeg,), jnp.float32),
        ],
    )
    def deg_kernel(dst_hbm, out_hbm, idx_v, ones_v, zbuf_v, acc_sh):
        cid = lax.axis_index("c")
        sid = lax.axis_index("s")

        def fill_ones(i, c):
            ones_v[pl.ds(i * 16, 16)] = jnp.full((16,), 1.0, jnp.float32)
            return c

        lax.fori_loop(0, K // 16, fill_ones, 0)

        def fill_zero(i, c):
            zbuf_v[pl.ds(i * 16, 16)] = jnp.zeros((16,), jnp.float32)
            return c

        lax.fori_loop(0, per // 16, fill_zero, 0)
        pltpu.sync_copy(zbuf_v, acc_sh.at[pl.ds(sid * per, per)])
        plsc.subcore_barrier()

        def body(j, c):
            g = j // G
            r = j % G

            @pl.when(r == 0)
            def _():
                pltpu.sync_copy(dst_hbm.at[cid, sid, pl.ds(g * G, G)], idx_v)

            pltpu.sync_copy(ones_v, acc_sh.at[idx_v.at[r]], add=True)
            return c

        lax.fori_loop(0, C, body, 0)
        plsc.subcore_barrier()
        pltpu.sync_copy(acc_sh.at[pl.ds(sid * per, per)],
                        out_hbm.at[cid, 0, pl.ds(sid * per, per)])

    return deg_kernel(dst_p)


@functools.lru_cache(maxsize=None)
def _sc_scatter_build(n, d, C, nacc):
    zper = nacc // NS
    mesh = plsc.VectorSubcoreMesh(core_axis_name="c", subcore_axis_name="s")

    @functools.partial(
        pl.kernel,
        out_type=jax.ShapeDtypeStruct((NC, nacc, d), jnp.float32),
        mesh=mesh,
        scratch_types=[
            pltpu.VMEM((G, K), jnp.int32),
            pltpu.VMEM((G, K), jnp.int32),
            pltpu.VMEM((K, d), jnp.float32),
            pltpu.VMEM_SHARED((nacc, d), jnp.float32),
            pltpu.SemaphoreType.DMA,
        ],
    )
    def scat_kernel(hs_hbm, src_hbm, dst_hbm, out_hbm,
                    sidx_v, didx_v, rows_v, acc_sh, sem):
        cid = lax.axis_index("c")
        sid = lax.axis_index("s")

        def fill_zero(t, c):
            rows_v[t // (d // 16), pl.ds((t % (d // 16)) * 16, 16)] = (
                jnp.zeros((16,), jnp.float32))
            return c

        lax.fori_loop(0, K * d // 16, fill_zero, 0)

        def zero_acc(t, c):
            pltpu.sync_copy(rows_v, acc_sh.at[pl.ds(sid * zper + t * K, K)])
            return c

        lax.fori_loop(0, zper // K, zero_acc, 0)
        plsc.subcore_barrier()

        def body(j, c):
            g = j // G
            r = j % G

            @pl.when(r == 0)
            def _():
                pltpu.sync_copy(src_hbm.at[cid, sid, pl.ds(g * G, G)], sidx_v)
                pltpu.sync_copy(dst_hbm.at[cid, sid, pl.ds(g * G, G)], didx_v)

            pltpu.async_copy(hs_hbm.at[sidx_v.at[r]], rows_v, sem).wait()
            pltpu.sync_copy(rows_v, acc_sh.at[didx_v.at[r]], add=True)
            return c

        lax.fori_loop(0, C, body, 0)
        plsc.subcore_barrier()

        def writeout(t, c):
            base = sid * zper + t * K
            pltpu.sync_copy(acc_sh.at[pl.ds(base, K)],
                            out_hbm.at[cid, pl.ds(base, K)])
            return c

        lax.fori_loop(0, zper // K, writeout, 0)

    return scat_kernel


def _sc_scatter(hs, src_p, dst_p, nacc):
    return _sc_scatter_build(hs.shape[0], hs.shape[1], src_p.shape[2], nacc)(
        hs, src_p, dst_p)


def _tc_dinv(deg_p):
    def body(p_ref, o_ref):
        o_ref[...] = lax.rsqrt(1.0 + jnp.sum(p_ref[...], axis=0, keepdims=True))

    return pl.pallas_call(
        body,
        out_shape=jax.ShapeDtypeStruct((1, deg_p.shape[1]), jnp.float32),
    )(deg_p)


def _tc_matmul_scale(x, w, dinv):
    n, d = x.shape
    h = w.shape[1]
    br = 1000

    def body(x_ref, w_ref, d_ref, o_ref):
        o_ref[...] = jnp.dot(x_ref[...], w_ref[...],
                             preferred_element_type=jnp.float32) * d_ref[...]

    return pl.pallas_call(
        body,
        grid=(n // br,),
        in_specs=[
            pl.BlockSpec((br, d), lambda i: (i, 0)),
            pl.BlockSpec((d, h), lambda i: (0, 0)),
            pl.BlockSpec((br, 1), lambda i: (i, 0)),
        ],
        out_specs=pl.BlockSpec((br, h), lambda i: (i, 0)),
        out_shape=jax.ShapeDtypeStruct((n, h), jnp.float32),
    )(x, w, dinv)


def _tc_mid(parts, hs, dinv, b, w):
    n, d = hs.shape
    h = w.shape[1]
    br = 1000

    def body(p_ref, hs_ref, d_ref, b_ref, w_ref, o_ref):
        s = p_ref[0] + p_ref[1] + hs_ref[...]
        z = jnp.maximum(s * d_ref[...] + b_ref[...], 0.0)
        o_ref[...] = jnp.dot(z, w_ref[...],
                             preferred_element_type=jnp.float32) * d_ref[...]

    return pl.pallas_call(
        body,
        grid=(n // br,),
        in_specs=[
            pl.BlockSpec((NC, br, d), lambda i: (0, i, 0)),
            pl.BlockSpec((br, d), lambda i: (i, 0)),
            pl.BlockSpec((br, 1), lambda i: (i, 0)),
            pl.BlockSpec((1, d), lambda i: (0, 0)),
            pl.BlockSpec((d, h), lambda i: (0, 0)),
        ],
        out_specs=pl.BlockSpec((br, h), lambda i: (i, 0)),
        out_shape=jax.ShapeDtypeStruct((n, h), jnp.float32),
    )(parts, hs, dinv, b, w)


def _tc_final(parts, hs, dinv, b, wl, bl):
    n, d = hs.shape
    out = wl.shape[1]
    br = 1000
    g = n // br

    def body(p_ref, hs_ref, d_ref, b_ref, wl_ref, bl_ref, o_ref, acc_ref):
        i = pl.program_id(0)

        @pl.when(i == 0)
        def _():
            acc_ref[...] = jnp.zeros_like(acc_ref)

        s = p_ref[0] + p_ref[1] + hs_ref[...]
        z = jnp.maximum(s * d_ref[...] + b_ref[...], 0.0)
        acc_ref[...] += jnp.sum(z, axis=0, keepdims=True)

        @pl.when(i == g - 1)
        def _():
            o_ref[...] = jnp.dot(acc_ref[...] * (1.0 / n), wl_ref[...],
                                 preferred_element_type=jnp.float32) + bl_ref[...]

    return pl.pallas_call(
        body,
        grid=(g,),
        in_specs=[
            pl.BlockSpec((NC, br, d), lambda i: (0, i, 0)),
            pl.BlockSpec((br, d), lambda i: (i, 0)),
            pl.BlockSpec((br, 1), lambda i: (i, 0)),
            pl.BlockSpec((1, d), lambda i: (0, 0)),
            pl.BlockSpec((d, out), lambda i: (0, 0)),
            pl.BlockSpec((1, out), lambda i: (0, 0)),
        ],
        out_specs=pl.BlockSpec((1, out), lambda i: (0, 0)),
        out_shape=jax.ShapeDtypeStruct((1, out), jnp.float32),
        scratch_shapes=[pltpu.VMEM((1, d), jnp.float32)],
    )(parts, hs, dinv, b, wl, bl)


def kernel(x, edge_index, W1, b1, W2, b2, Wl, bl):
    n, d = x.shape
    e = edge_index.shape[1]
    c_per_tile = -(-e // (NC * NS * K * G)) * G
    e_pad = NC * NS * c_per_tile * K
    pad = e_pad - e

    nacc = ((n + 16) + (NS * K) - 1) // (NS * K) * (NS * K)
    ndeg = nacc
    ar = jnp.arange(pad, dtype=jnp.int32)
    src_p = jnp.concatenate([edge_index[0], (ar * 97) % n])
    dst_p = jnp.concatenate([edge_index[1], n + ar % (nacc - n)])
    src_p = src_p.reshape(NC, NS, c_per_tile, K)
    dst_p = dst_p.reshape(NC, NS, c_per_tile, K)

    deg_p = _sc_degree(dst_p, ndeg).reshape(NC, ndeg)
    dinv = _tc_dinv(deg_p).reshape(ndeg, 1)[:n]

    hs1 = _tc_matmul_scale(x, W1, dinv)
    p1 = _sc_scatter(hs1, src_p, dst_p, nacc)
    hs2 = _tc_mid(p1, hs1, dinv, b1.reshape(1, -1), W2)
    p2 = _sc_scatter(hs2, src_p, dst_p, nacc)
    res = _tc_final(p2, hs2, dinv, b2.reshape(1, -1), Wl, bl.reshape(1, -1))
    return res.reshape(-1)


# speedup vs baseline: 31.5243x; 1.4645x over previous
import functools

import jax
import jax.numpy as jnp
from jax import lax
from jax.experimental import pallas as pl
from jax.experimental.pallas import tpu as pltpu
from jax.experimental.pallas import tpu_sc as plsc

NC = 2
NS = 16
K = 128
G = 8


def _sc_degree(dst_p, ndeg):
    C = dst_p.shape[2]
    per = ndeg // NS
    mesh = plsc.VectorSubcoreMesh(core_axis_name="c", subcore_axis_name="s")

    @functools.partial(
        pl.kernel,
        out_type=jax.ShapeDtypeStruct((NC, 1, ndeg), jnp.float32),
        mesh=mesh,
        scratch_types=[
            pltpu.VMEM((G, K), jnp.int32),
            pltpu.VMEM((K,), jnp.float32),
            pltpu.VMEM((per,), jnp.float32),
            pltpu.VMEM_SHARED((ndeg,), jnp.float32),
        ],
    )
    def deg_kernel(dst_hbm, out_hbm, idx_v, ones_v, zbuf_v, acc_sh):
        cid = lax.axis_index("c")
        sid = lax.axis_index("s")

        def fill_ones(i, c):
            ones_v[pl.ds(i * 16, 16)] = jnp.full((16,), 1.0, jnp.float32)
            return c

        lax.fori_loop(0, K // 16, fill_ones, 0)

        def fill_zero(i, c):
            zbuf_v[pl.ds(i * 16, 16)] = jnp.zeros((16,), jnp.float32)
            return c

        lax.fori_loop(0, per // 16, fill_zero, 0)
        pltpu.sync_copy(zbuf_v, acc_sh.at[pl.ds(sid * per, per)])
        plsc.subcore_barrier()

        def body(j, c):
            g = j // G
            r = j % G

            @pl.when(r == 0)
            def _():
                pltpu.sync_copy(dst_hbm.at[cid, sid, pl.ds(g * G, G)], idx_v)

            pltpu.sync_copy(ones_v, acc_sh.at[idx_v.at[r]], add=True)
            return c

        lax.fori_loop(0, C, body, 0)
        plsc.subcore_barrier()
        pltpu.sync_copy(acc_sh.at[pl.ds(sid * per, per)],
                        out_hbm.at[cid, 0, pl.ds(sid * per, per)])

    return deg_kernel(dst_p)


@functools.lru_cache(maxsize=None)
def _sc_scatter_build(n, d, C, nacc):
    zper = nacc // NS
    mesh = plsc.VectorSubcoreMesh(core_axis_name="c", subcore_axis_name="s")

    @functools.partial(
        pl.kernel,
        out_type=jax.ShapeDtypeStruct((NC, nacc, d), jnp.float32),
        mesh=mesh,
        scratch_types=[
            pltpu.VMEM((2, G, K), jnp.int32),
            pltpu.VMEM((2, G, K), jnp.int32),
            pltpu.VMEM((2, K, d), jnp.float32),
            pltpu.VMEM_SHARED((nacc, d), jnp.float32),
            pltpu.SemaphoreType.DMA((2,)),
            pltpu.SemaphoreType.DMA((2,)),
        ],
    )
    def scat_kernel(hs_hbm, src_hbm, dst_hbm, out_hbm,
                    sidx_v, didx_v, rows_v, acc_sh, gsem, ssem):
        cid = lax.axis_index("c")
        sid = lax.axis_index("s")

        def fill_zero(t, c):
            rows_v[0, t // (d // 16), pl.ds((t % (d // 16)) * 16, 16)] = (
                jnp.zeros((16,), jnp.float32))
            return c

        lax.fori_loop(0, K * d // 16, fill_zero, 0)

        def zero_acc_start(t, c):
            pltpu.async_copy(rows_v.at[0],
                             acc_sh.at[pl.ds(sid * zper + t * K, K)],
                             gsem.at[0])
            return c

        lax.fori_loop(0, zper // K, zero_acc_start, 0)

        def zero_acc_wait(t, c):
            pltpu.make_async_copy(
                rows_v.at[0], acc_sh.at[pl.ds(sid * zper + t * K, K)],
                gsem.at[0]).wait()
            return c

        lax.fori_loop(0, zper // K, zero_acc_wait, 0)
        plsc.subcore_barrier()

        def gather_start(j):
            pltpu.async_copy(
                hs_hbm.at[sidx_v.at[(j // G) % 2, j % G]],
                rows_v.at[j % 2], gsem.at[j % 2])

        def scat_descr(j):
            return pltpu.make_async_copy(
                rows_v.at[j % 2],
                acc_sh.at[didx_v.at[(j // G) % 2, j % G]],
                ssem.at[j % 2])

        pltpu.sync_copy(src_hbm.at[cid, sid, pl.ds(0, G)], sidx_v.at[0])
        pltpu.sync_copy(dst_hbm.at[cid, sid, pl.ds(0, G)], didx_v.at[0])
        gather_start(0)

        def body(j, c):
            g = j // G
            r = j % G

            @pl.when(jnp.logical_and(r == 0, g + 1 < C // G))
            def _():
                pltpu.sync_copy(src_hbm.at[cid, sid, pl.ds((g + 1) * G, G)],
                                sidx_v.at[(g + 1) % 2])
                pltpu.sync_copy(dst_hbm.at[cid, sid, pl.ds((g + 1) * G, G)],
                                didx_v.at[(g + 1) % 2])

            @pl.when(j >= 1)
            def _():
                scat_descr(j - 1).wait()

            @pl.when(j + 1 < C)
            def _():
                gather_start(j + 1)

            pltpu.make_async_copy(
                hs_hbm.at[sidx_v.at[g % 2, r]], rows_v.at[j % 2],
                gsem.at[j % 2]).wait()
            scat_descr(j).start(add=True)
            return c

        lax.fori_loop(0, C, body, 0)
        scat_descr(C - 1).wait()
        plsc.subcore_barrier()

        def writeout_start(t, c):
            base = sid * zper + t * K
            pltpu.async_copy(acc_sh.at[pl.ds(base, K)],
                             out_hbm.at[cid, pl.ds(base, K)], gsem.at[0])
            return c

        lax.fori_loop(0, zper // K, writeout_start, 0)

        def writeout_wait(t, c):
            base = sid * zper + t * K
            pltpu.make_async_copy(acc_sh.at[pl.ds(base, K)],
                                  out_hbm.at[cid, pl.ds(base, K)],
                                  gsem.at[0]).wait()
            return c

        lax.fori_loop(0, zper // K, writeout_wait, 0)

    return scat_kernel


def _sc_scatter(hs, src_p, dst_p, nacc):
    return _sc_scatter_build(hs.shape[0], hs.shape[1], src_p.shape[2], nacc)(
        hs, src_p, dst_p)


def _tc_dinv(deg_p):
    def body(p_ref, o_ref):
        o_ref[...] = lax.rsqrt(1.0 + jnp.sum(p_ref[...], axis=0, keepdims=True))

    return pl.pallas_call(
        body,
        out_shape=jax.ShapeDtypeStruct((1, deg_p.shape[1]), jnp.float32),
    )(deg_p)


def _tc_matmul_scale(x, w, dinv):
    n, d = x.shape
    h = w.shape[1]
    br = 1000

    def body(x_ref, w_ref, d_ref, o_ref):
        o_ref[...] = jnp.dot(x_ref[...], w_ref[...],
                             preferred_element_type=jnp.float32) * d_ref[...]

    return pl.pallas_call(
        body,
        grid=(n // br,),
        in_specs=[
            pl.BlockSpec((br, d), lambda i: (i, 0)),
            pl.BlockSpec((d, h), lambda i: (0, 0)),
            pl.BlockSpec((br, 1), lambda i: (i, 0)),
        ],
        out_specs=pl.BlockSpec((br, h), lambda i: (i, 0)),
        out_shape=jax.ShapeDtypeStruct((n, h), jnp.float32),
    )(x, w, dinv)


def _tc_mid(parts, hs, dinv, b, w):
    n, d = hs.shape
    h = w.shape[1]
    br = 1000

    def body(p_ref, hs_ref, d_ref, b_ref, w_ref, o_ref):
        s = p_ref[0] + p_ref[1] + hs_ref[...]
        z = jnp.maximum(s * d_ref[...] + b_ref[...], 0.0)
        o_ref[...] = jnp.dot(z, w_ref[...],
                             preferred_element_type=jnp.float32) * d_ref[...]

    return pl.pallas_call(
        body,
        grid=(n // br,),
        in_specs=[
            pl.BlockSpec((NC, br, d), lambda i: (0, i, 0)),
            pl.BlockSpec((br, d), lambda i: (i, 0)),
            pl.BlockSpec((br, 1), lambda i: (i, 0)),
            pl.BlockSpec((1, d), lambda i: (0, 0)),
            pl.BlockSpec((d, h), lambda i: (0, 0)),
        ],
        out_specs=pl.BlockSpec((br, h), lambda i: (i, 0)),
        out_shape=jax.ShapeDtypeStruct((n, h), jnp.float32),
    )(parts, hs, dinv, b, w)


def _tc_final(parts, hs, dinv, b, wl, bl):
    n, d = hs.shape
    out = wl.shape[1]
    br = 1000
    g = n // br

    def body(p_ref, hs_ref, d_ref, b_ref, wl_ref, bl_ref, o_ref, acc_ref):
        i = pl.program_id(0)

        @pl.when(i == 0)
        def _():
            acc_ref[...] = jnp.zeros_like(acc_ref)

        s = p_ref[0] + p_ref[1] + hs_ref[...]
        z = jnp.maximum(s * d_ref[...] + b_ref[...], 0.0)
        acc_ref[...] += jnp.sum(z, axis=0, keepdims=True)

        @pl.when(i == g - 1)
        def _():
            o_ref[...] = jnp.dot(acc_ref[...] * (1.0 / n), wl_ref[...],
                                 preferred_element_type=jnp.float32) + bl_ref[...]

    return pl.pallas_call(
        body,
        grid=(g,),
        in_specs=[
            pl.BlockSpec((NC, br, d), lambda i: (0, i, 0)),
            pl.BlockSpec((br, d), lambda i: (i, 0)),
            pl.BlockSpec((br, 1), lambda i: (i, 0)),
            pl.BlockSpec((1, d), lambda i: (0, 0)),
            pl.BlockSpec((d, out), lambda i: (0, 0)),
            pl.BlockSpec((1, out), lambda i: (0, 0)),
        ],
        out_specs=pl.BlockSpec((1, out), lambda i: (0, 0)),
        out_shape=jax.ShapeDtypeStruct((1, out), jnp.float32),
        scratch_shapes=[pltpu.VMEM((1, d), jnp.float32)],
    )(parts, hs, dinv, b, wl, bl)


def kernel(x, edge_index, W1, b1, W2, b2, Wl, bl):
    n, d = x.shape
    e = edge_index.shape[1]
    c_per_tile = -(-e // (NC * NS * K * G)) * G
    e_pad = NC * NS * c_per_tile * K
    pad = e_pad - e

    nacc = ((n + 16) + (NS * K) - 1) // (NS * K) * (NS * K)
    ndeg = nacc
    ar = jnp.arange(pad, dtype=jnp.int32)
    src_p = jnp.concatenate([edge_index[0], (ar * 97) % n])
    dst_p = jnp.concatenate([edge_index[1], n + ar % (nacc - n)])
    src_p = src_p.reshape(NC, NS, c_per_tile, K)
    dst_p = dst_p.reshape(NC, NS, c_per_tile, K)

    deg_p = _sc_degree(dst_p, ndeg).reshape(NC, ndeg)
    dinv = _tc_dinv(deg_p).reshape(ndeg, 1)[:n]

    hs1 = _tc_matmul_scale(x, W1, dinv)
    p1 = _sc_scatter(hs1, src_p, dst_p, nacc)
    hs2 = _tc_mid(p1, hs1, dinv, b1.reshape(1, -1), W2)
    p2 = _sc_scatter(hs2, src_p, dst_p, nacc)
    res = _tc_final(p2, hs2, dinv, b2.reshape(1, -1), Wl, bl.reshape(1, -1))
    return res.reshape(-1)


# speedup vs baseline: 32.4015x; 1.0278x over previous
import functools

import jax
import jax.numpy as jnp
from jax import lax
from jax.experimental import pallas as pl
from jax.experimental.pallas import tpu as pltpu
from jax.experimental.pallas import tpu_sc as plsc

NC = 2
NS = 16
K = 128
G = 8


def _sc_degree(dst_p, ndeg):
    C = dst_p.shape[2]
    per = ndeg // NS
    mesh = plsc.VectorSubcoreMesh(core_axis_name="c", subcore_axis_name="s")

    @functools.partial(
        pl.kernel,
        out_type=jax.ShapeDtypeStruct((NC, 1, ndeg), jnp.float32),
        mesh=mesh,
        scratch_types=[
            pltpu.VMEM((G, K), jnp.int32),
            pltpu.VMEM((K,), jnp.float32),
            pltpu.VMEM((per,), jnp.float32),
            pltpu.VMEM_SHARED((ndeg,), jnp.float32),
        ],
    )
    def deg_kernel(dst_hbm, out_hbm, idx_v, ones_v, zbuf_v, acc_sh):
        cid = lax.axis_index("c")
        sid = lax.axis_index("s")

        def fill_ones(i, c):
            ones_v[pl.ds(i * 16, 16)] = jnp.full((16,), 1.0, jnp.float32)
            return c

        lax.fori_loop(0, K // 16, fill_ones, 0)

        def fill_zero(i, c):
            zbuf_v[pl.ds(i * 16, 16)] = jnp.zeros((16,), jnp.float32)
            return c

        lax.fori_loop(0, per // 16, fill_zero, 0)
        pltpu.sync_copy(zbuf_v, acc_sh.at[pl.ds(sid * per, per)])
        plsc.subcore_barrier()

        def body(j, c):
            g = j // G
            r = j % G

            @pl.when(r == 0)
            def _():
                pltpu.sync_copy(dst_hbm.at[cid, sid, pl.ds(g * G, G)], idx_v)

            pltpu.sync_copy(ones_v, acc_sh.at[idx_v.at[r]], add=True)
            return c

        lax.fori_loop(0, C, body, 0)
        plsc.subcore_barrier()
        pltpu.sync_copy(acc_sh.at[pl.ds(sid * per, per)],
                        out_hbm.at[cid, 0, pl.ds(sid * per, per)])

    return deg_kernel(dst_p)


@functools.lru_cache(maxsize=None)
def _sc_scatter_build(n, d, C, nacc):
    zper = nacc // NS
    mesh = plsc.VectorSubcoreMesh(core_axis_name="c", subcore_axis_name="s")

    @functools.partial(
        pl.kernel,
        out_type=jax.ShapeDtypeStruct((NC, nacc, d), jnp.float32),
        mesh=mesh,
        scratch_types=[
            pltpu.VMEM((2, G, K), jnp.int32),
            pltpu.VMEM((2, G, K), jnp.int32),
            pltpu.VMEM((2, K, d), jnp.float32),
            pltpu.VMEM_SHARED((nacc, d), jnp.float32),
            pltpu.SemaphoreType.DMA((2,)),
            pltpu.SemaphoreType.DMA((2,)),
        ],
    )
    def scat_kernel(hs_hbm, src_hbm, dst_hbm, out_hbm,
                    sidx_v, didx_v, rows_v, acc_sh, gsem, ssem):
        cid = lax.axis_index("c")
        sid = lax.axis_index("s")

        def fill_zero(t, c):
            rows_v[0, t // (d // 16), pl.ds((t % (d // 16)) * 16, 16)] = (
                jnp.zeros((16,), jnp.float32))
            return c

        lax.fori_loop(0, K * d // 16, fill_zero, 0)

        def zero_acc_start(t, c):
            pltpu.async_copy(rows_v.at[0],
                             acc_sh.at[pl.ds(sid * zper + t * K, K)],
                             gsem.at[0])
            return c

        lax.fori_loop(0, zper // K, zero_acc_start, 0)

        def zero_acc_wait(t, c):
            pltpu.make_async_copy(
                rows_v.at[0], acc_sh.at[pl.ds(sid * zper + t * K, K)],
                gsem.at[0]).wait()
            return c

        lax.fori_loop(0, zper // K, zero_acc_wait, 0)
        plsc.subcore_barrier()

        def gather_start(j):
            pltpu.async_copy(
                hs_hbm.at[sidx_v.at[(j // G) % 2, j % G]],
                rows_v.at[j % 2], gsem.at[j % 2])

        def scat_descr(j):
            return pltpu.make_async_copy(
                rows_v.at[j % 2],
                acc_sh.at[didx_v.at[(j // G) % 2, j % G]],
                ssem.at[j % 2])

        pltpu.sync_copy(src_hbm.at[cid, sid, pl.ds(0, G)], sidx_v.at[0])
        pltpu.sync_copy(dst_hbm.at[cid, sid, pl.ds(0, G)], didx_v.at[0])
        gather_start(0)

        def body(j, c):
            g = j // G
            r = j % G

            @pl.when(jnp.logical_and(r == 0, g + 1 < C // G))
            def _():
                pltpu.sync_copy(src_hbm.at[cid, sid, pl.ds((g + 1) * G, G)],
                                sidx_v.at[(g + 1) % 2])
                pltpu.sync_copy(dst_hbm.at[cid, sid, pl.ds((g + 1) * G, G)],
                                didx_v.at[(g + 1) % 2])

            @pl.when(j >= 1)
            def _():
                scat_descr(j - 1).wait()

            @pl.when(j + 1 < C)
            def _():
                gather_start(j + 1)

            pltpu.make_async_copy(
                hs_hbm.at[sidx_v.at[g % 2, r]], rows_v.at[j % 2],
                gsem.at[j % 2]).wait()
            scat_descr(j).start(add=True)
            return c

        lax.fori_loop(0, C, body, 0)
        scat_descr(C - 1).wait()
        plsc.subcore_barrier()

        def writeout_start(t, c):
            base = sid * zper + t * K
            pltpu.async_copy(acc_sh.at[pl.ds(base, K)],
                             out_hbm.at[cid, pl.ds(base, K)], gsem.at[0])
            return c

        lax.fori_loop(0, zper // K, writeout_start, 0)

        def writeout_wait(t, c):
            base = sid * zper + t * K
            pltpu.make_async_copy(acc_sh.at[pl.ds(base, K)],
                                  out_hbm.at[cid, pl.ds(base, K)],
                                  gsem.at[0]).wait()
            return c

        lax.fori_loop(0, zper // K, writeout_wait, 0)

    return scat_kernel


def _sc_scatter(hs, src_p, dst_p, nacc):
    return _sc_scatter_build(hs.shape[0], hs.shape[1], src_p.shape[2], nacc)(
        hs, src_p, dst_p)


def _tc_matmul(x, w):
    n, d = x.shape
    h = w.shape[1]
    br = 1000

    def body(x_ref, w_ref, o_ref):
        o_ref[...] = jnp.dot(x_ref[...], w_ref[...],
                             preferred_element_type=jnp.float32)

    return pl.pallas_call(
        body,
        grid=(n // br,),
        in_specs=[
            pl.BlockSpec((br, d), lambda i: (i, 0)),
            pl.BlockSpec((d, h), lambda i: (0, 0)),
        ],
        out_specs=pl.BlockSpec((br, h), lambda i: (i, 0)),
        out_shape=jax.ShapeDtypeStruct((n, h), jnp.float32),
    )(x, w)


def _tc_scale(h1, deg_p):
    n, h = h1.shape
    ndeg = deg_p.shape[1]

    def body(p_ref, h_ref, o_ref, d_ref):
        dr = lax.rsqrt(1.0 + jnp.sum(p_ref[...], axis=0, keepdims=True))
        dcol = jnp.transpose(dr)[:n]
        d_ref[...] = dcol
        o_ref[...] = h_ref[...] * dcol

    return pl.pallas_call(
        body,
        out_shape=[
            jax.ShapeDtypeStruct((n, h), jnp.float32),
            jax.ShapeDtypeStruct((n, 1), jnp.float32),
        ],
    )(deg_p, h1)


def _tc_mid(parts, hs, dinv, b, w):
    n, d = hs.shape
    h = w.shape[1]
    br = 1000

    def body(p_ref, hs_ref, d_ref, b_ref, w_ref, o_ref):
        s = p_ref[0] + p_ref[1] + hs_ref[...]
        z = jnp.maximum(s * d_ref[...] + b_ref[...], 0.0)
        o_ref[...] = jnp.dot(z, w_ref[...],
                             preferred_element_type=jnp.float32) * d_ref[...]

    return pl.pallas_call(
        body,
        grid=(n // br,),
        in_specs=[
            pl.BlockSpec((NC, br, d), lambda i: (0, i, 0)),
            pl.BlockSpec((br, d), lambda i: (i, 0)),
            pl.BlockSpec((br, 1), lambda i: (i, 0)),
            pl.BlockSpec((1, d), lambda i: (0, 0)),
            pl.BlockSpec((d, h), lambda i: (0, 0)),
        ],
        out_specs=pl.BlockSpec((br, h), lambda i: (i, 0)),
        out_shape=jax.ShapeDtypeStruct((n, h), jnp.float32),
    )(parts, hs, dinv, b, w)


def _tc_final(parts, hs, dinv, b, wl, bl):
    n, d = hs.shape
    out = wl.shape[1]
    br = 1000
    g = n // br

    def body(p_ref, hs_ref, d_ref, b_ref, wl_ref, bl_ref, o_ref, acc_ref):
        i = pl.program_id(0)

        @pl.when(i == 0)
        def _():
            acc_ref[...] = jnp.zeros_like(acc_ref)

        s = p_ref[0] + p_ref[1] + hs_ref[...]
        z = jnp.maximum(s * d_ref[...] + b_ref[...], 0.0)
        acc_ref[...] += jnp.sum(z, axis=0, keepdims=True)

        @pl.when(i == g - 1)
        def _():
            o_ref[...] = jnp.dot(acc_ref[...] * (1.0 / n), wl_ref[...],
                                 preferred_element_type=jnp.float32) + bl_ref[...]

    return pl.pallas_call(
        body,
        grid=(g,),
        in_specs=[
            pl.BlockSpec((NC, br, d), lambda i: (0, i, 0)),
            pl.BlockSpec((br, d), lambda i: (i, 0)),
            pl.BlockSpec((br, 1), lambda i: (i, 0)),
            pl.BlockSpec((1, d), lambda i: (0, 0)),
            pl.BlockSpec((d, out), lambda i: (0, 0)),
            pl.BlockSpec((1, out), lambda i: (0, 0)),
        ],
        out_specs=pl.BlockSpec((1, out), lambda i: (0, 0)),
        out_shape=jax.ShapeDtypeStruct((1, out), jnp.float32),
        scratch_shapes=[pltpu.VMEM((1, d), jnp.float32)],
    )(parts, hs, dinv, b, wl, bl)


def kernel(x, edge_index, W1, b1, W2, b2, Wl, bl):
    n, d = x.shape
    e = edge_index.shape[1]
    c_per_tile = -(-e // (NC * NS * K * G)) * G
    e_pad = NC * NS * c_per_tile * K
    pad = e_pad - e

    nacc = ((n + 16) + (NS * K) - 1) // (NS * K) * (NS * K)
    ndeg = nacc
    ar = jnp.arange(pad, dtype=jnp.int32)
    src_p = jnp.concatenate([edge_index[0], (ar * 97) % n])
    dst_p = jnp.concatenate([edge_index[1], n + ar % (nacc - n)])
    src_p = src_p.reshape(NC, NS, c_per_tile, K)
    dst_p = dst_p.reshape(NC, NS, c_per_tile, K)

    deg_p = _sc_degree(dst_p, ndeg).reshape(NC, ndeg)
    h1 = _tc_matmul(x, W1)
    hs1, dinv = _tc_scale(h1, deg_p)
    p1 = _sc_scatter(hs1, src_p, dst_p, nacc)
    hs2 = _tc_mid(p1, hs1, dinv, b1.reshape(1, -1), W2)
    p2 = _sc_scatter(hs2, src_p, dst_p, nacc)
    res = _tc_final(p2, hs2, dinv, b2.reshape(1, -1), Wl, bl.reshape(1, -1))
    return res.reshape(-1)
